# baseline (speedup 1.0000x reference)
import functools

import jax
import jax.numpy as jnp
from jax import lax
from jax.experimental import pallas as pl
from jax.experimental.pallas import tpu as pltpu
from jax.experimental.pallas import tpu_sc as plsc

BATCH = 16384
HIST = 200
EMB_DIM = 32

NUM_WORKERS = 32
B_PER_W = BATCH // NUM_WORKERS
CHUNK = B_PER_W
N_CHUNKS = HIST
NBUF = 4
K = 3
NSTG = 2
LANES = 16
assert N_CHUNKS % NBUF == 0 and 0 < K < NBUF and NBUF % NSTG == 0


def _emb_kernel(idx_hbm, table_hbm, out_hbm, *scratch):
    idx_v = scratch[:NBUF]
    rows_v = scratch[NBUF : 2 * NBUF]
    stg_v = scratch[2 * NBUF : 2 * NBUF + NSTG]
    isem, gsem, ssem = scratch[2 * NBUF + NSTG :]

    wid = lax.axis_index("s") * 2 + lax.axis_index("c")
    b0 = wid * B_PER_W

    def idx_start(i, b):
        pltpu.async_copy(
            idx_hbm.at[pl.ds(i * BATCH + b0, CHUNK)], idx_v[b], isem.at[b]
        )

    def idx_wait(b):
        pltpu.make_async_copy(
            idx_hbm.at[pl.ds(b0, CHUNK)], idx_v[b], isem.at[b]
        ).wait()

    def gather_start(b):
        pltpu.async_copy(table_hbm.at[idx_v[b]], rows_v[b], gsem.at[b])

    def gather_wait(b):
        pltpu.make_async_copy(
            table_hbm.at[idx_v[b]], rows_v[b], gsem.at[b]
        ).wait()

    def store_start(j, s):
        pltpu.async_copy(
            stg_v[s], out_hbm.at[j, :, pl.ds(b0, CHUNK)], ssem.at[s]
        )

    def store_wait(s):
        pltpu.make_async_copy(
            stg_v[s], out_hbm.at[0, :, pl.ds(b0, CHUNK)], ssem.at[s]
        ).wait()

    def transpose(b1, s):
        rows, stg = rows_v[b1], stg_v[s]

        @pl.loop(0, CHUNK // LANES)
        def _t(kk):
            row_idx = lax.iota(jnp.int32, LANES) + kk * LANES
            off = kk * LANES
            for e in range(EMB_DIM):
                col_idx = jnp.full((LANES,), e, jnp.int32)
                stg[e, pl.ds(off, LANES)] = plsc.load_gather(
                    rows, [row_idx, col_idx]
                )

    def finish(j, b1, s, prefetch, swait):
        gather_wait(b1)
        if swait:
            store_wait(s)
        transpose(b1, s)
        store_start(j, s)
        if prefetch:
            idx_start(j + NBUF, b1)

    for b in range(NBUF):
        idx_start(b, b)

    for i in range(2 * NBUF):
        b = i % NBUF
        idx_wait(b)
        gather_start(b)
        j = i - K
        if j >= 0:
            finish(j, j % NBUF, j % NSTG, prefetch=True, swait=j >= NSTG)

    @pl.loop(2 * NBUF, N_CHUNKS - NBUF, step=NBUF)
    def _steady(g):
        for b in range(NBUF):
            i = g + b
            j = i - K
            idx_wait(b)
            gather_start(b)
            finish(j, (b - K) % NBUF, (b - K) % NSTG, True, True)

    for b in range(NBUF):
        i = N_CHUNKS - NBUF + b
        idx_wait(b)
        gather_start(b)
        j = i - K
        finish(j, j % NBUF, j % NSTG, prefetch=j + NBUF < N_CHUNKS, swait=True)

    for j in range(N_CHUNKS - K, N_CHUNKS):
        finish(j, j % NBUF, j % NSTG, prefetch=False, swait=True)
    for s in range(NSTG):
        store_wait(s)


@jax.jit
def _embedding_lookup(xt_flat, table):
    mesh = plsc.VectorSubcoreMesh(core_axis_name="c", subcore_axis_name="s")
    scratch = (
        [pltpu.VMEM((CHUNK,), jnp.int32) for _ in range(NBUF)]
        + [pltpu.VMEM((CHUNK, EMB_DIM), jnp.float32) for _ in range(NBUF)]
        + [pltpu.VMEM((EMB_DIM, CHUNK), jnp.float32) for _ in range(NSTG)]
        + [
            pltpu.SemaphoreType.DMA((NBUF,)),
            pltpu.SemaphoreType.DMA((NBUF,)),
            pltpu.SemaphoreType.DMA((NSTG,)),
        ]
    )
    k = functools.partial(
        pl.kernel,
        out_type=jax.ShapeDtypeStruct((HIST, EMB_DIM, BATCH), jnp.float32),
        mesh=mesh,
        scratch_types=scratch,
        compiler_params=pltpu.CompilerParams(
            use_tc_tiling_on_sc=False, needs_layout_passes=False
        ),
    )(_emb_kernel)
    return k(xt_flat, table)


def kernel(x, table):
    xt_flat = x.T.reshape(-1).astype(jnp.int32)
    out_t = _embedding_lookup(xt_flat, table)
    return jnp.transpose(out_t, (2, 0, 1))


# speedup vs baseline: 1.5843x; 1.5843x over previous
import functools

import jax
import jax.numpy as jnp
from jax import lax
from jax.experimental import pallas as pl
from jax.experimental.pallas import tpu as pltpu
from jax.experimental.pallas import tpu_sc as plsc

BATCH = 16384
HIST = 200
EMB_DIM = 32

NUM_WORKERS = 32
B_PER_W = BATCH // NUM_WORKERS
CHUNK = B_PER_W
N_CHUNKS = HIST
NBUF = 5
K = 3
assert N_CHUNKS % NBUF == 0 and 0 < K < NBUF


def _emb_kernel(idx_hbm, table_hbm, out_hbm, idx_v, rows_v, isem, gsem, ssem):
    wid = lax.axis_index("s") * 2 + lax.axis_index("c")
    b0 = wid * B_PER_W

    def idx_start(i, b):
        pltpu.async_copy(
            idx_hbm.at[i, pl.ds(b0, CHUNK)], idx_v.at[b], isem.at[b]
        )

    def idx_wait(b):
        pltpu.make_async_copy(
            idx_hbm.at[0, pl.ds(b0, CHUNK)], idx_v.at[b], isem.at[b]
        ).wait()

    def gather_start(b):
        pltpu.async_copy(table_hbm.at[idx_v.at[b]], rows_v.at[b], gsem.at[b])

    def gather_wait(b):
        pltpu.make_async_copy(
            table_hbm.at[idx_v.at[b]], rows_v.at[b], gsem.at[b]
        ).wait()

    def store_start(i, b):
        pltpu.async_copy(
            rows_v.at[b], out_hbm.at[i, pl.ds(b0, CHUNK)], ssem.at[b]
        )

    def store_wait(b):
        pltpu.make_async_copy(
            rows_v.at[b], out_hbm.at[0, pl.ds(b0, CHUNK)], ssem.at[b]
        ).wait()

    def finish(j, b1, prefetch):
        gather_wait(b1)
        store_start(j, b1)
        if prefetch:
            idx_start(j + NBUF, b1)

    for b in range(NBUF):
        idx_start(b, b)

    for b in range(NBUF):
        idx_wait(b)
        gather_start(b)
        j = b - K
        if j >= 0:
            finish(j, j % NBUF, prefetch=True)

    @pl.loop(NBUF, N_CHUNKS - NBUF, step=NBUF)
    def _steady(g):
        for b in range(NBUF):
            i = g + b
            store_wait(b)
            idx_wait(b)
            gather_start(b)
            finish(i - K, (b - K) % NBUF, prefetch=True)

    for b in range(NBUF):
        i = N_CHUNKS - NBUF + b
        store_wait(b)
        idx_wait(b)
        gather_start(b)
        j = i - K
        finish(j, j % NBUF, prefetch=j + NBUF < N_CHUNKS)

    for j in range(N_CHUNKS - K, N_CHUNKS):
        finish(j, j % NBUF, prefetch=False)
    for b in range(NBUF):
        store_wait(b)


@jax.jit
def _embedding_lookup(xt, table):
    mesh = plsc.VectorSubcoreMesh(core_axis_name="c", subcore_axis_name="s")
    k = functools.partial(
        pl.kernel,
        out_type=jax.ShapeDtypeStruct((HIST, BATCH, EMB_DIM), jnp.float32),
        mesh=mesh,
        scratch_types=[
            pltpu.VMEM((NBUF, CHUNK), jnp.int32),
            pltpu.VMEM((NBUF, CHUNK, EMB_DIM), jnp.float32),
            pltpu.SemaphoreType.DMA((NBUF,)),
            pltpu.SemaphoreType.DMA((NBUF,)),
            pltpu.SemaphoreType.DMA((NBUF,)),
        ],
        compiler_params=pltpu.CompilerParams(use_tc_tiling_on_sc=False),
    )(_emb_kernel)
    return k(xt, table)


def kernel(x, table):
    xt = x.T.astype(jnp.int32)
    out_t = _embedding_lookup(xt, table)
    return jnp.transpose(out_t, (1, 0, 2))


# speedup vs baseline: 1.9125x; 1.2072x over previous
import functools

import jax
import jax.numpy as jnp
from jax import lax
from jax.experimental import pallas as pl
from jax.experimental.pallas import tpu as pltpu
from jax.experimental.pallas import tpu_sc as plsc

BATCH = 16384
HIST = 200
EMB_DIM = 32

NUM_WORKERS = 32
B_PER_W = BATCH // NUM_WORKERS
CHUNK = B_PER_W
N_CHUNKS = HIST
NBUF = 4
K = 3
NSTG = 2
LANES = 16
assert N_CHUNKS % NBUF == 0 and 0 < K < NBUF and NBUF % NSTG == 0


def _emb_kernel(idx_hbm, table_hbm, out_hbm, *scratch):
    idx_v = scratch[:NBUF]
    rows_v = scratch[NBUF : 2 * NBUF]
    stg_v = scratch[2 * NBUF : 2 * NBUF + NSTG]
    isem, gsem, ssem = scratch[2 * NBUF + NSTG :]

    wid = lax.axis_index("s") * 2 + lax.axis_index("c")
    b0 = wid * B_PER_W
    iota = lax.iota(jnp.int32, LANES)
    perms = [(iota + d) & (LANES - 1) for d in range(LANES)]

    def idx_start(i, b):
        pltpu.async_copy(
            idx_hbm.at[i, pl.ds(b0, CHUNK)], idx_v[b], isem.at[b]
        )

    def idx_wait(b):
        pltpu.make_async_copy(
            idx_hbm.at[0, pl.ds(b0, CHUNK)], idx_v[b], isem.at[b]
        ).wait()

    def gather_start(b):
        pltpu.async_copy(table_hbm.at[idx_v[b]], rows_v[b], gsem.at[b])

    def gather_wait(b):
        pltpu.make_async_copy(
            table_hbm.at[idx_v[b]], rows_v[b], gsem.at[b]
        ).wait()

    def store_start(j, s):
        pltpu.async_copy(
            stg_v[s], out_hbm.at[j, :, pl.ds(b0, CHUNK)], ssem.at[s]
        )

    def store_wait(s):
        pltpu.make_async_copy(
            stg_v[s], out_hbm.at[0, :, pl.ds(b0, CHUNK)], ssem.at[s]
        ).wait()

    def transpose(b1, s):
        rows, stg = rows_v[b1], stg_v[s]

        @pl.loop(0, CHUNK // LANES)
        def _t(rb):
            row_vec = iota + rb * LANES
            for eo in range(EMB_DIM // LANES):
                for d in range(LANES):
                    e_vec = perms[d] + (eo * LANES)
                    vals = plsc.load_gather(rows, [row_vec, e_vec])
                    plsc.store_scatter(stg, [e_vec, row_vec], vals)

    def finish(j, b1, s, prefetch, swait):
        gather_wait(b1)
        if swait:
            store_wait(s)
        transpose(b1, s)
        store_start(j, s)
        if prefetch:
            idx_start(j + NBUF, b1)

    for b in range(NBUF):
        idx_start(b, b)

    for i in range(2 * NBUF):
        b = i % NBUF
        idx_wait(b)
        gather_start(b)
        j = i - K
        if j >= 0:
            finish(j, j % NBUF, j % NSTG, prefetch=True, swait=j >= NSTG)

    @pl.loop(2 * NBUF, N_CHUNKS - NBUF, step=NBUF)
    def _steady(g):
        for b in range(NBUF):
            i = g + b
            j = i - K
            idx_wait(b)
            gather_start(b)
            finish(j, (b - K) % NBUF, (b - K) % NSTG, True, True)

    for b in range(NBUF):
        i = N_CHUNKS - NBUF + b
        idx_wait(b)
        gather_start(b)
        j = i - K
        finish(j, j % NBUF, j % NSTG, prefetch=j + NBUF < N_CHUNKS, swait=True)

    for j in range(N_CHUNKS - K, N_CHUNKS):
        finish(j, j % NBUF, j % NSTG, prefetch=False, swait=True)
    for s in range(NSTG):
        store_wait(s)


@jax.jit
def _embedding_lookup(xt, table):
    mesh = plsc.VectorSubcoreMesh(core_axis_name="c", subcore_axis_name="s")
    scratch = (
        [pltpu.VMEM((CHUNK,), jnp.int32) for _ in range(NBUF)]
        + [pltpu.VMEM((CHUNK, EMB_DIM), jnp.float32) for _ in range(NBUF)]
        + [pltpu.VMEM((EMB_DIM, CHUNK), jnp.float32) for _ in range(NSTG)]
        + [
            pltpu.SemaphoreType.DMA((NBUF,)),
            pltpu.SemaphoreType.DMA((NBUF,)),
            pltpu.SemaphoreType.DMA((NSTG,)),
        ]
    )
    k = functools.partial(
        pl.kernel,
        out_type=jax.ShapeDtypeStruct((HIST, EMB_DIM, BATCH), jnp.float32),
        mesh=mesh,
        scratch_types=scratch,
        compiler_params=pltpu.CompilerParams(
            use_tc_tiling_on_sc=False, needs_layout_passes=False
        ),
    )(_emb_kernel)
    return k(xt, table)


def kernel(x, table):
    xt = x.T.astype(jnp.int32)
    out_t = _embedding_lookup(xt, table)
    return jnp.transpose(out_t, (2, 0, 1))


# speedup vs baseline: 1.9792x; 1.0348x over previous
import functools

import jax
import jax.numpy as jnp
from jax import lax
from jax.experimental import pallas as pl
from jax.experimental.pallas import tpu as pltpu
from jax.experimental.pallas import tpu_sc as plsc

BATCH = 16384
HIST = 200
EMB_DIM = 32

NUM_WORKERS = 32
B_PER_W = BATCH // NUM_WORKERS
CHUNK = B_PER_W
N_CHUNKS = HIST
NBUF = 4
K = 3
NSTG = 2
LANES = 16
assert N_CHUNKS % NBUF == 0 and 0 < K < NBUF and NBUF % NSTG == 0


def _emb_kernel(idx_hbm, table_hbm, out_hbm, *scratch):
    idx_v = scratch[:NBUF]
    rows_v = scratch[NBUF : 2 * NBUF]
    stg_v = scratch[2 * NBUF : 2 * NBUF + NSTG]
    isem, gsem, ssem = scratch[2 * NBUF + NSTG :]

    wid = lax.axis_index("s") * 2 + lax.axis_index("c")
    b0 = wid * B_PER_W
    iota = lax.iota(jnp.int32, LANES)
    perms = [
        ((iota + d) & (LANES - 1)) + eo * LANES
        for eo in range(EMB_DIM // LANES)
        for d in range(LANES)
    ]

    def idx_start(i, b):
        pltpu.async_copy(
            idx_hbm.at[i, pl.ds(b0, CHUNK)], idx_v[b], isem.at[b]
        )

    def idx_wait(b):
        pltpu.make_async_copy(
            idx_hbm.at[0, pl.ds(b0, CHUNK)], idx_v[b], isem.at[b]
        ).wait()

    def gather_start(b):
        pltpu.async_copy(table_hbm.at[idx_v[b]], rows_v[b], gsem.at[b])

    def gather_wait(b):
        pltpu.make_async_copy(
            table_hbm.at[idx_v[b]], rows_v[b], gsem.at[b]
        ).wait()

    def store_start(j, s):
        pltpu.async_copy(
            stg_v[s], out_hbm.at[j, :, pl.ds(b0, CHUNK)], ssem.at[s]
        )

    def store_wait(s):
        pltpu.make_async_copy(
            stg_v[s], out_hbm.at[0, :, pl.ds(b0, CHUNK)], ssem.at[s]
        ).wait()

    def transpose(b1, s):
        rows, stg = rows_v[b1], stg_v[s]

        @pl.loop(0, CHUNK // LANES)
        def _t(rb):
            row_vec = iota + rb * LANES
            for e_vec in perms:
                vals = plsc.load_gather(rows, [row_vec, e_vec])
                plsc.store_scatter(stg, [e_vec, row_vec], vals)

    def finish(j, b1, s, prefetch, swait):
        gather_wait(b1)
        if swait:
            store_wait(s)
        transpose(b1, s)
        store_start(j, s)
        if prefetch:
            idx_start(j + NBUF, b1)

    for b in range(NBUF):
        idx_start(b, b)

    for i in range(2 * NBUF):
        b = i % NBUF
        idx_wait(b)
        gather_start(b)
        j = i - K
        if j >= 0:
            finish(j, j % NBUF, j % NSTG, prefetch=True, swait=j >= NSTG)

    @pl.loop(2 * NBUF, N_CHUNKS - NBUF, step=NBUF)
    def _steady(g):
        for b in range(NBUF):
            i = g + b
            j = i - K
            idx_wait(b)
            gather_start(b)
            finish(j, (b - K) % NBUF, (b - K) % NSTG, True, True)

    for b in range(NBUF):
        i = N_CHUNKS - NBUF + b
        idx_wait(b)
        gather_start(b)
        j = i - K
        finish(j, j % NBUF, j % NSTG, prefetch=j + NBUF < N_CHUNKS, swait=True)

    for j in range(N_CHUNKS - K, N_CHUNKS):
        finish(j, j % NBUF, j % NSTG, prefetch=False, swait=True)
    for s in range(NSTG):
        store_wait(s)


@jax.jit
def _embedding_lookup(xt, table):
    mesh = plsc.VectorSubcoreMesh(core_axis_name="c", subcore_axis_name="s")
    scratch = (
        [pltpu.VMEM((CHUNK,), jnp.int32) for _ in range(NBUF)]
        + [pltpu.VMEM((CHUNK, EMB_DIM), jnp.float32) for _ in range(NBUF)]
        + [pltpu.VMEM((EMB_DIM, CHUNK), jnp.float32) for _ in range(NSTG)]
        + [
            pltpu.SemaphoreType.DMA((NBUF,)),
            pltpu.SemaphoreType.DMA((NBUF,)),
            pltpu.SemaphoreType.DMA((NSTG,)),
        ]
    )
    k = functools.partial(
        pl.kernel,
        out_type=jax.ShapeDtypeStruct((HIST, EMB_DIM, BATCH), jnp.float32),
        mesh=mesh,
        scratch_types=scratch,
        compiler_params=pltpu.CompilerParams(
            use_tc_tiling_on_sc=False, needs_layout_passes=False
        ),
    )(_emb_kernel)
    return k(xt, table)


def kernel(x, table):
    xt = x.T.astype(jnp.int32)
    out_t = _embedding_lookup(xt, table)
    return jnp.transpose(out_t, (2, 0, 1))
